# Initial kernel scaffold; baseline (speedup 1.0000x reference)
import jax
import jax.numpy as jnp
from jax.experimental import pallas as pl


def kernel(x, edge_index, batch, lin0_W, lin0_b, conv1_Ws, conv1_b, conv2_Ws, conv2_b, lin1_W, lin1_b, lin2_W, lin2_b, lin3_W, lin3_b):
    raise NotImplementedError("write your pallas kernel here")


# speedup vs baseline: 5.2735x; 5.2735x over previous
import functools

import jax
import jax.numpy as jnp
from jax import lax
from jax.experimental import pallas as pl
from jax.experimental.pallas import tpu as pltpu
from jax.experimental.pallas import tpu_sc as plsc

N = 10000
NP = 10240
F = 128
NC, NS = 2, 16
NW = NC * NS
C = 128
BN = 1000
ROWS_PER_TILE = NP // NS

_mesh = plsc.VectorSubcoreMesh(core_axis_name="c", subcore_axis_name="s")


def _zero_vmem(buf, n_rows, width):
    def zbody(k, carry):
        i = k // (width // 16)
        j = k - i * (width // 16)
        buf[i, pl.ds(j * 16, 16)] = jnp.zeros((16,), jnp.float32)
        return carry
    lax.fori_loop(0, n_rows * (width // 16), zbody, 0)


def _make_prop(G):
    @functools.partial(
        pl.kernel,
        out_type=jax.ShapeDtypeStruct((NC, NP, F), jnp.float32),
        mesh=_mesh,
        scratch_types=[
            pltpu.VMEM((C,), jnp.int32),
            pltpu.VMEM((C,), jnp.int32),
            pltpu.VMEM((C, F), jnp.float32),
            pltpu.VMEM_SHARED((NP, F), jnp.float32),
            pltpu.SemaphoreType.DMA,
        ],
    )
    def _prop(row_hbm, col_hbm, g_hbm, out_hbm, idx_row, idx_col, rows_v, acc_sh, sem):
        cid = lax.axis_index("c")
        sid = lax.axis_index("s")
        wid = sid * NC + cid

        _zero_vmem(rows_v, C, F)
        zb = sid * ROWS_PER_TILE
        for m in range(ROWS_PER_TILE // C):
            pltpu.sync_copy(rows_v, acc_sh.at[pl.ds(zb + m * C, C)])
        plsc.subcore_barrier()

        ebase = wid * (G * C)

        def body(g, carry):
            off = pl.multiple_of(ebase + g * C, 8)
            pltpu.sync_copy(row_hbm.at[pl.ds(off, C)], idx_row)
            pltpu.sync_copy(col_hbm.at[pl.ds(off, C)], idx_col)
            pltpu.async_copy(g_hbm.at[idx_row], rows_v, sem).wait()
            pltpu.sync_copy(rows_v, acc_sh.at[idx_col], add=True)
            return carry

        lax.fori_loop(0, G, body, 0)

        plsc.subcore_barrier()
        rb = sid * ROWS_PER_TILE
        pltpu.sync_copy(acc_sh.at[pl.ds(rb, ROWS_PER_TILE)],
                        out_hbm.at[cid, pl.ds(rb, ROWS_PER_TILE)])

    return _prop


def _make_deg(G):
    @functools.partial(
        pl.kernel,
        out_type=jax.ShapeDtypeStruct((NC, NP, F), jnp.float32),
        mesh=_mesh,
        scratch_types=[
            pltpu.VMEM((C,), jnp.int32),
            pltpu.VMEM((C, F), jnp.float32),
            pltpu.VMEM_SHARED((NP, F), jnp.float32),
        ],
    )
    def _deg(col_hbm, out_hbm, idx_col, ones_v, acc_sh):
        cid = lax.axis_index("c")
        sid = lax.axis_index("s")
        wid = sid * NC + cid

        _zero_vmem(ones_v, C, F)
        zb = sid * ROWS_PER_TILE
        for m in range(ROWS_PER_TILE // C):
            pltpu.sync_copy(ones_v, acc_sh.at[pl.ds(zb + m * C, C)])

        def obody(k, carry):
            i = k // (F // 16)
            j = k - i * (F // 16)
            ones_v[i, pl.ds(j * 16, 16)] = jnp.ones((16,), jnp.float32)
            return carry
        lax.fori_loop(0, C * (F // 16), obody, 0)
        plsc.subcore_barrier()

        ebase = wid * (G * C)

        def body(g, carry):
            off = pl.multiple_of(ebase + g * C, 8)
            pltpu.sync_copy(col_hbm.at[pl.ds(off, C)], idx_col)
            pltpu.sync_copy(ones_v, acc_sh.at[idx_col], add=True)
            return carry

        lax.fori_loop(0, G, body, 0)

        plsc.subcore_barrier()
        rb = sid * ROWS_PER_TILE
        pltpu.sync_copy(acc_sh.at[pl.ds(rb, ROWS_PER_TILE)],
                        out_hbm.at[cid, pl.ds(rb, ROWS_PER_TILE)])

    return _deg


_row_spec = pl.BlockSpec((BN, F), lambda i: (i, 0))
_w_spec = pl.BlockSpec((F, F), lambda i: (0, 0))
_b_spec = pl.BlockSpec((1, F), lambda i: (0, 0))
_GRID = N // BN


def _tc_call(body, n_out):
    outs = tuple(jax.ShapeDtypeStruct((N, F), jnp.float32) for _ in range(n_out))
    return lambda specs, *args: pl.pallas_call(
        body,
        grid=(_GRID,),
        in_specs=list(specs),
        out_specs=tuple(_row_spec for _ in range(n_out)),
        out_shape=outs,
    )(*args)


def _pre_body(x_ref, w0_ref, b0_ref, ws_ref, dga_ref, dgb_ref,
              acc_ref, g_ref, dis_ref):
    h = jnp.maximum(x_ref[...] @ w0_ref[...] + b0_ref[...], 0.0)
    d = dga_ref[:, 0:1] + dgb_ref[:, 0:1]
    dis = jnp.where(d > 0.0, lax.rsqrt(d), 0.0)
    acc_ref[...] = h @ ws_ref[...]
    g_ref[...] = dis * h
    dis_ref[...] = jnp.broadcast_to(dis, (BN, F))


def _step_body(sa_ref, sb_ref, dis_ref, acc_ref, wk_ref, acc_out, g_out):
    dis = dis_ref[...]
    h = dis * (sa_ref[...] + sb_ref[...])
    acc_out[...] = acc_ref[...] + h @ wk_ref[...]
    g_out[...] = dis * h


def _last_body(sa_ref, sb_ref, dis_ref, acc_ref, wk_ref, b_ref, wn_ref,
               acc2_out, g_out):
    dis = dis_ref[...]
    h = dis * (sa_ref[...] + sb_ref[...])
    c = jnp.maximum(acc_ref[...] + h @ wk_ref[...] + b_ref[...], 0.0)
    acc2_out[...] = c @ wn_ref[...]
    g_out[...] = dis * c


def _tail_body(sa_ref, sb_ref, dis_ref, acc_ref, wk_ref, b_ref,
               l1w_ref, l1b_ref, l2w_ref, l2b_ref, l3w_ref, l3b_ref, y_out):
    dis = dis_ref[...]
    h = dis * (sa_ref[...] + sb_ref[...])
    c = jnp.maximum(acc_ref[...] + h @ wk_ref[...] + b_ref[...], 0.0)
    t = jnp.maximum(c @ l1w_ref[...] + l1b_ref[...], 0.0)
    t = jnp.maximum(t @ l2w_ref[...] + l2b_ref[...], 0.0)
    y_out[...] = jnp.maximum(t @ l3w_ref[...] + l3b_ref[...], 0.0)


def kernel(x, edge_index, batch, lin0_W, lin0_b, conv1_Ws, conv1_b,
           conv2_Ws, conv2_b, lin1_W, lin1_b, lin2_W, lin2_b, lin3_W, lin3_b):
    E = edge_index.shape[1]
    G = -(-E // (NW * C))
    EP = G * NW * C
    pad = EP - E
    row = jnp.concatenate([edge_index[0], jnp.zeros((pad,), jnp.int32)])
    col = jnp.concatenate([edge_index[1], jnp.full((pad,), N, jnp.int32)])

    prop = _make_prop(G)
    degk = _make_deg(G)

    degp = degk(col)

    dg_spec = _row_spec
    b2 = lambda v: v.reshape(1, F)

    acc, g, dis_b = _tc_call(_pre_body, 3)(
        [_row_spec, _w_spec, _b_spec, _w_spec, dg_spec, dg_spec],
        x, lin0_W, b2(lin0_b), conv1_Ws[0], degp[0], degp[1])

    def hop(acc, g, wk):
        s = prop(row, col, g)
        return _tc_call(_step_body, 2)(
            [_row_spec, _row_spec, _row_spec, _row_spec, _w_spec],
            s[0], s[1], dis_b, acc, wk)

    acc, g = hop(acc, g, conv1_Ws[1])
    acc, g = hop(acc, g, conv1_Ws[2])
    s = prop(row, col, g)
    acc, g = _tc_call(_last_body, 2)(
        [_row_spec, _row_spec, _row_spec, _row_spec, _w_spec, _b_spec, _w_spec],
        s[0], s[1], dis_b, acc, conv1_Ws[3], b2(conv1_b), conv2_Ws[0])

    acc, g = hop(acc, g, conv2_Ws[1])
    acc, g = hop(acc, g, conv2_Ws[2])
    s = prop(row, col, g)

    y = pl.pallas_call(
        _tail_body,
        grid=(_GRID,),
        in_specs=[_row_spec, _row_spec, _row_spec, _row_spec, _w_spec, _b_spec,
                  _w_spec, _b_spec, _w_spec, _b_spec,
                  pl.BlockSpec((F, 1), lambda i: (0, 0)),
                  pl.BlockSpec((1, 1), lambda i: (0, 0))],
        out_specs=pl.BlockSpec((BN, 1), lambda i: (i, 0)),
        out_shape=jax.ShapeDtypeStruct((N, 1), jnp.float32),
    )(s[0], s[1], dis_b, acc, conv2_Ws[3], b2(conv2_b),
      lin1_W, b2(lin1_b), lin2_W, b2(lin2_b), lin3_W, lin3_b.reshape(1, 1))
    return y
